# baseline (speedup 1.0000x reference)
import functools

import jax
import jax.numpy as jnp
from jax import lax
from jax.experimental import pallas as pl
from jax.experimental.pallas import tpu as pltpu
from jax.experimental.pallas import tpu_sc as plsc

_VOCAB = 100000
_DIM = 64
_K = 4
_N = 16384

_NC = 2
_NS = 16
_NW = _NC * _NS
_BPW = _N // _NW
_XCH = 4
_XB = _BPW // _XCH


_LANE = 16


@jax.jit
def _fire_word_gather(ranks, func_weight, measure_x, measure_m):
    mesh = plsc.VectorSubcoreMesh(core_axis_name="c", subcore_axis_name="s")
    mm16 = measure_m.reshape(_VOCAB // 4, 16)

    @functools.partial(
        pl.kernel,
        mesh=mesh,
        compiler_params=pltpu.CompilerParams(use_tc_tiling_on_sc=False,
                                              needs_layout_passes=False),
        out_type=(
            jax.ShapeDtypeStruct((_N, _DIM), jnp.float32),
            jax.ShapeDtypeStruct((_N, _K, _DIM), jnp.float32),
            jax.ShapeDtypeStruct((_N, _K), jnp.float32),
        ),
        scratch_types=[
            pltpu.VMEM((_BPW,), jnp.int32),
            pltpu.VMEM((_BPW,), jnp.int32),
            pltpu.VMEM((_BPW, _DIM), jnp.float32),
            pltpu.VMEM((2, _XB, _K, _DIM), jnp.float32),
            pltpu.VMEM((_BPW, 16), jnp.float32),
            pltpu.VMEM((_BPW, _K), jnp.float32),
            pltpu.SemaphoreType.DMA,
            pltpu.SemaphoreType.DMA,
            pltpu.SemaphoreType.DMA,
            pltpu.SemaphoreType.DMA,
        ],
    )
    def k(ranks_hbm, fw_hbm, mx_hbm, mm_hbm,
          out_f_hbm, out_x_hbm, out_m_hbm,
          idx_v, idxq_v, f_v, x_v, g_v, m_v,
          sem_f, sem_m, sem_x0, sem_x1):
        wid = lax.axis_index("s") * _NC + lax.axis_index("c")
        base = wid * _BPW
        pltpu.sync_copy(ranks_hbm.at[pl.ds(base, _BPW)], idx_v)
        cf = pltpu.async_copy(fw_hbm.at[idx_v], f_v, sem_f)
        for i in range(_BPW // _LANE):
            r = idx_v[pl.ds(i * _LANE, _LANE)]
            idxq_v[pl.ds(i * _LANE, _LANE)] = lax.shift_right_logical(r, 2)
        cm = pltpu.async_copy(mm_hbm.at[idxq_v], g_v, sem_m)
        sems = (sem_x0, sem_x1)
        copies = [None, None]
        copies[0] = pltpu.async_copy(
            mx_hbm.at[idx_v.at[pl.ds(0, _XB)]], x_v.at[0], sems[0])
        for i in range(_XCH):
            if i + 1 < _XCH:
                copies[(i + 1) % 2] = pltpu.async_copy(
                    mx_hbm.at[idx_v.at[pl.ds((i + 1) * _XB, _XB)]],
                    x_v.at[(i + 1) % 2], sems[(i + 1) % 2])
            copies[i % 2].wait()
            pltpu.sync_copy(x_v.at[i % 2],
                            out_x_hbm.at[pl.ds(base + i * _XB, _XB)])
        cf.wait()
        pltpu.sync_copy(f_v, out_f_hbm.at[pl.ds(base, _BPW)])
        cm.wait()
        lanes = lax.iota(jnp.int32, _LANE)
        for i in range(_BPW // _LANE):
            jvec = lanes + i * _LANE
            r = idx_v[pl.ds(i * _LANE, _LANE)]
            off = lax.shift_left(lax.bitwise_and(r, 3), 2)
            for t in range(_K):
                vals = plsc.load_gather(g_v, [jvec, off + t])
                plsc.store_scatter(m_v, [jvec, lanes * 0 + t], vals)
        pltpu.sync_copy(m_v, out_m_hbm.at[pl.ds(base, _BPW)])

    return k(ranks, func_weight, measure_x, mm16)


def kernel(ranks, func_weight, measure_x, measure_m):
    return _fire_word_gather(ranks, func_weight, measure_x, measure_m)
